# baseline (speedup 1.0000x reference)
import functools

import jax
import jax.numpy as jnp
from jax import lax
from jax.experimental import pallas as pl
from jax.experimental.pallas import tpu as pltpu
from jax.experimental.pallas import tpu_sc as plsc

NUM_CLASSES = 1000000
BATCH = 16384

_INFO = plsc.get_sparse_core_info()
_NC, _NS = _INFO.num_cores, _INFO.num_subcores
_NW = _NC * _NS
_BPW = BATCH // _NW
_GCHUNK = 128
_NGC = _BPW // _GCHUNK

_GRANULES = 3906
_PER_S = _GRANULES * 16
_TAIL_OFF = _PER_S * _NS
_TAIL = NUM_CLASSES - _TAIL_OFF
_NCHUNK = 4
_CGRAN = 976
_CHUNK_ELEMS = _CGRAN * 16
_UNROLL = 8

_LN2 = 0.6931471805599453
_C = 0.7071067811865476


def _lad_log(v):
    k = jnp.zeros(v.shape, jnp.float32)
    for p in (64, 32, 16, 8, 4, 2, 1):
        m = 2.0 ** p
        cond = v * m < 2.0 * _C
        v = jnp.where(cond, v * m, v)
        k = jnp.where(cond, k - float(p), k)
    for p in (64, 32, 16, 8, 4, 2, 1):
        m = 2.0 ** -p
        cond = v * m >= _C
        v = jnp.where(cond, v * m, v)
        k = jnp.where(cond, k + float(p), k)
    s = (v - 1.0) / (v + 1.0)
    s2 = s * s
    t = 2.0 * s * (1.0 + s2 * (1.0 / 3.0 + s2 * (0.2 + s2 * (1.0 / 7.0))))
    return t + k * _LN2


def _sc_kernel(probs, idx):
    mesh = plsc.VectorSubcoreMesh(core_axis_name="c", subcore_axis_name="s")

    @functools.partial(
        pl.kernel,
        mesh=mesh,
        out_type=(
            jax.ShapeDtypeStruct((BATCH,), jnp.float32),
            jax.ShapeDtypeStruct((16,), jnp.float32),
        ),
        scratch_types=[
            pltpu.VMEM((_NGC, _GCHUNK), jnp.int32),
            pltpu.VMEM((_BPW,), jnp.float32),
            pltpu.VMEM((_PER_S,), jnp.float32),
            pltpu.VMEM((_TAIL,), jnp.float32),
            pltpu.VMEM((16,), jnp.float32),
            pltpu.VMEM((16,), jnp.int32),
            pltpu.VMEM((16,), jnp.float32),
            pltpu.VMEM_SHARED((16,), jnp.float32),
            pltpu.SemaphoreType.DMA,
            pltpu.SemaphoreType.DMA((_NCHUNK,)),
        ],
    )
    def k(table_hbm, idx_hbm, zvec_hbm, out_hbm, psum_hbm,
          idx_v, vals_v, slab_v, tail_v, part_v, izeros_v, splat_v, shared_tot,
          gsem, csem):
        cid = lax.axis_index("c")
        sid = lax.axis_index("s")
        wid = sid * _NC + cid
        gbase = wid * _BPW
        sbase = sid * _PER_S

        chunk_cps = [
            pltpu.async_copy(
                table_hbm.at[pl.ds(sbase + c * _CHUNK_ELEMS, _CHUNK_ELEMS)],
                slab_v.at[pl.ds(c * _CHUNK_ELEMS, _CHUNK_ELEMS)],
                csem.at[c])
            for c in range(_NCHUNK)
        ]
        rest_cp = pltpu.async_copy(
            table_hbm.at[pl.ds(sbase + _NCHUNK * _CHUNK_ELEMS,
                               (_GRANULES - _NCHUNK * _CGRAN) * 16)],
            slab_v.at[pl.ds(_NCHUNK * _CHUNK_ELEMS,
                            (_GRANULES - _NCHUNK * _CGRAN) * 16)],
            gsem)

        for j in range(_NGC):
            pltpu.sync_copy(idx_hbm.at[pl.ds(gbase + j * _GCHUNK, _GCHUNK)],
                            idx_v.at[j])
        gather_cps = [
            pltpu.async_copy(table_hbm.at[idx_v.at[j]],
                             vals_v.at[pl.ds(j * _GCHUNK, _GCHUNK)], gsem)
            for j in range(_NGC)
        ]

        pltpu.sync_copy(zvec_hbm, izeros_v)

        @pl.when(sid == 0)
        def _zero_total():
            splat_v[...] = jnp.zeros((16,), jnp.float32)
            pltpu.sync_copy(splat_v, shared_tot)

        zeros = jnp.zeros((16,), jnp.float32)
        accs = [zeros] * _UNROLL
        for c in range(_NCHUNK):
            chunk_cps[c].wait()
            cbase = c * _CHUNK_ELEMS

            def body(i, a, _cbase=cbase):
                base = _cbase + i * (16 * _UNROLL)
                return tuple(
                    a[u] + slab_v[pl.ds(base + u * 16, 16)]
                    for u in range(_UNROLL)
                )

            accs = lax.fori_loop(0, _CGRAN // _UNROLL, body, tuple(accs))
        acc = accs[0]
        for u in range(1, _UNROLL):
            acc = acc + accs[u]
        rest_cp.wait()
        for g in range(_NCHUNK * _CGRAN, _GRANULES):
            acc = acc + slab_v[pl.ds(g * 16, 16)]

        part_v[...] = acc

        @pl.when(sid == 0)
        def _add_tail():
            pltpu.sync_copy(table_hbm.at[pl.ds(_TAIL_OFF, _TAIL)], tail_v)
            extra = jnp.zeros((16,), jnp.float32)
            for g in range(_TAIL // 16):
                extra = extra + tail_v[pl.ds(g * 16, 16)]
            part_v[...] = acc + extra

        plsc.subcore_barrier()
        pltpu.sync_copy(part_v, shared_tot.at[izeros_v], add=True)
        plsc.subcore_barrier()

        @pl.when(sid == 0)
        def _publish_total():
            pltpu.sync_copy(shared_tot, psum_hbm)

        plsc.subcore_barrier()
        pltpu.async_copy(psum_hbm.at[izeros_v], splat_v, gsem).wait()
        log_total = _lad_log(splat_v[...])

        for cp in gather_cps:
            cp.wait()
        for g in range(_BPW // 16):
            v = vals_v[pl.ds(g * 16, 16)]
            vals_v[pl.ds(g * 16, 16)] = _lad_log(v) - log_total
        pltpu.sync_copy(vals_v, out_hbm.at[pl.ds(gbase, _BPW)])

    out, _ = k(probs, idx, jnp.zeros((16,), jnp.int32))
    return out


def kernel(probs, x):
    idx = x.reshape(BATCH).astype(jnp.int32)
    return _sc_kernel(probs, idx)


# speedup vs baseline: 1.3446x; 1.3446x over previous
import functools

import jax
import jax.numpy as jnp
from jax import lax
from jax.experimental import pallas as pl
from jax.experimental.pallas import tpu as pltpu
from jax.experimental.pallas import tpu_sc as plsc

NUM_CLASSES = 1000000
BATCH = 16384

_INFO = plsc.get_sparse_core_info()
_NC, _NS = _INFO.num_cores, _INFO.num_subcores
_NW = _NC * _NS
_BPW = BATCH // _NW
_GCHUNK = 128
_NGC = _BPW // _GCHUNK

_GRANULES = 1953
_PER_W = _GRANULES * 16
_TAIL_OFF = _PER_W * _NW
_TAIL = NUM_CLASSES - _TAIL_OFF
_NCHUNK = 4
_CGRAN = 488
_CHUNK_ELEMS = _CGRAN * 16
_UNROLL = 8


def _sc_gather_sum(probs, idx):
    mesh = plsc.VectorSubcoreMesh(core_axis_name="c", subcore_axis_name="s")

    @functools.partial(
        pl.kernel,
        mesh=mesh,
        out_type=(
            jax.ShapeDtypeStruct((BATCH,), jnp.float32),
            jax.ShapeDtypeStruct((_NW * 16,), jnp.float32),
        ),
        scratch_types=[
            pltpu.VMEM((_BPW,), jnp.int32),
            pltpu.VMEM((_BPW,), jnp.float32),
            pltpu.VMEM((_PER_W,), jnp.float32),
            pltpu.VMEM((_TAIL,), jnp.float32),
            pltpu.VMEM((16,), jnp.float32),
            pltpu.SemaphoreType.DMA,
            pltpu.SemaphoreType.DMA((_NCHUNK,)),
        ],
    )
    def k(table_hbm, idx_hbm, out_hbm, psum_hbm,
          idx_v, vals_v, slab_v, tail_v, part_v, gsem, csem):
        wid = lax.axis_index("s") * _NC + lax.axis_index("c")
        gbase = wid * _BPW
        sbase = wid * _PER_W

        chunk_cps = [
            pltpu.async_copy(
                table_hbm.at[pl.ds(sbase + c * _CHUNK_ELEMS, _CHUNK_ELEMS)],
                slab_v.at[pl.ds(c * _CHUNK_ELEMS, _CHUNK_ELEMS)],
                csem.at[c])
            for c in range(_NCHUNK)
        ]
        rest_cp = pltpu.async_copy(
            table_hbm.at[pl.ds(sbase + _NCHUNK * _CHUNK_ELEMS,
                               (_GRANULES - _NCHUNK * _CGRAN) * 16)],
            slab_v.at[pl.ds(_NCHUNK * _CHUNK_ELEMS,
                            (_GRANULES - _NCHUNK * _CGRAN) * 16)],
            gsem)

        pltpu.sync_copy(idx_hbm.at[pl.ds(gbase, _BPW)], idx_v)
        gather_cps = [
            pltpu.async_copy(table_hbm.at[idx_v.at[pl.ds(j * _GCHUNK, _GCHUNK)]],
                             vals_v.at[pl.ds(j * _GCHUNK, _GCHUNK)], gsem)
            for j in range(_NGC)
        ]

        zeros = jnp.zeros((16,), jnp.float32)
        accs = [zeros] * _UNROLL
        for c in range(_NCHUNK):
            chunk_cps[c].wait()
            cbase = c * _CHUNK_ELEMS

            def body(i, a, _cbase=cbase):
                base = _cbase + i * (16 * _UNROLL)
                return tuple(
                    a[u] + slab_v[pl.ds(base + u * 16, 16)]
                    for u in range(_UNROLL)
                )

            accs = lax.fori_loop(0, _CGRAN // _UNROLL, body, tuple(accs))
        acc = accs[0]
        for u in range(1, _UNROLL):
            acc = acc + accs[u]
        rest_cp.wait()
        for g in range(_NCHUNK * _CGRAN, _GRANULES):
            acc = acc + slab_v[pl.ds(g * 16, 16)]

        part_v[...] = acc

        @pl.when(wid == 0)
        def _add_tail():
            pltpu.sync_copy(table_hbm.at[pl.ds(_TAIL_OFF, _TAIL)], tail_v)
            extra = jnp.zeros((16,), jnp.float32)
            for g in range(_TAIL // 16):
                extra = extra + tail_v[pl.ds(g * 16, 16)]
            part_v[...] = acc + extra

        pltpu.sync_copy(part_v, psum_hbm.at[pl.ds(wid * 16, 16)])

        for j in range(_NGC):
            gather_cps[j].wait()
        pltpu.sync_copy(vals_v, out_hbm.at[pl.ds(gbase, _BPW)])

    return k(probs, idx)


def _tc_body(g_ref, p_ref, out_ref):
    total = jnp.sum(p_ref[...])
    out_ref[...] = jnp.log(g_ref[...]) - jnp.log(total)


def _tc_combine(gathered, psums):
    return pl.pallas_call(
        _tc_body,
        out_shape=jax.ShapeDtypeStruct((BATCH,), jnp.float32),
    )(gathered, psums)


def kernel(probs, x):
    idx = x.reshape(BATCH).astype(jnp.int32)
    gathered, psums = _sc_gather_sum(probs, idx)
    return _tc_combine(gathered, psums)
